# baseline (speedup 1.0000x reference)
import functools

import jax
import jax.numpy as jnp
from jax import lax
from jax.experimental import pallas as pl
from jax.experimental.pallas import tpu as pltpu
from jax.experimental.pallas import tpu_sc as plsc

N = 10000
E = 320000
D = 128
K = 16
ALPHA = 0.05
COEF = (1.0 - ALPHA) / K

NC = 2
NS = 16
NW = NC * NS

N2 = 10240
RPT = N2 // NS
SUP = 7
NSUP = 12
NSPAD = NSUP + 1
NCHUNK = SUP * NSUP
NCPAD = SUP * NSPAD
EPT = NCHUNK * 128
E2 = NW * EPT

_GATHER_DNUMS = jax.lax.GatherDimensionNumbers(
    offset_dims=(), collapsed_slice_dims=(0,), start_index_map=(0,))


def _lane_bcast(v16, e):
    idx = jnp.full((16, 1), e, dtype=jnp.int32)
    return jax.lax.gather(v16, idx, _GATHER_DNUMS, (1,),
                          mode=jax.lax.GatherScatterMode.PROMISE_IN_BOUNDS)


def _onehot(e):
    lanes = lax.iota(jnp.int32, 16)
    return jnp.where(lanes == e, jnp.float32(1.0), jnp.float32(0.0))


def _norm_body(dinvrep_hbm, eidx_hbm, ew_hbm, nrm_hbm,
               ebuf, wbuf, dr, dc, nout, sem):
    cid = lax.axis_index("c")
    sid = lax.axis_index("s")
    wid = cid * NS + sid

    def chunk_body(j, _):
        pltpu.sync_copy(eidx_hbm.at[wid, j // SUP, j % SUP], ebuf)
        pltpu.sync_copy(ew_hbm.at[wid, j], wbuf)
        pltpu.async_copy(dinvrep_hbm.at[ebuf.at[0]], dr, sem).wait()
        pltpu.async_copy(dinvrep_hbm.at[ebuf.at[1]], dc, sem).wait()

        def g_body(q, _):
            sl = pl.ds(q * 16, 16)
            nv = jnp.zeros((16,), jnp.float32)
            for e in range(16):
                k = q * 16 + e
                nv = nv + dr[k, :] * dc[k, :] * _onehot(e)
            nout[sl] = nv * wbuf[sl]
            return 0

        lax.fori_loop(0, 8, g_body, 0)
        pltpu.sync_copy(nout, nrm_hbm.at[wid, j])
        return 0

    lax.fori_loop(0, NCPAD, chunk_body, 0)


_norm = functools.partial(
    pl.kernel,
    out_type=jax.ShapeDtypeStruct((NW, NCPAD, 128), jnp.float32),
    mesh=plsc.VectorSubcoreMesh(core_axis_name="c", subcore_axis_name="s"),
    compiler_params=pltpu.CompilerParams(use_tc_tiling_on_sc=False),
    scratch_types=[
        pltpu.VMEM((2, 128), jnp.int32),
        pltpu.VMEM((128,), jnp.float32),
        pltpu.VMEM((128, 16), jnp.float32),
        pltpu.VMEM((128, 16), jnp.float32),
        pltpu.VMEM((128,), jnp.float32),
        pltpu.SemaphoreType.DMA,
    ],
)(_norm_body)


def _step_body(cur_hbm, eidx_hbm, nrm_hbm, zeros_hbm,
               p_hbm,
               ebuf, nbuf, rows, agg, semg, semi):
    cid = lax.axis_index("c")
    sid = lax.axis_index("s")
    wid = cid * NS + sid

    def refill(J, r):
        pltpu.async_copy(eidx_hbm.at[wid, J], ebuf.at[r], semi.at[r])
        pltpu.async_copy(nrm_hbm.at[wid, J], nbuf.at[r], semi.at[r])

    def wait_refill(r):
        pltpu.make_async_copy(eidx_hbm.at[wid, 0], ebuf.at[r],
                              semi.at[r]).wait()
        pltpu.make_async_copy(nrm_hbm.at[wid, 0], nbuf.at[r],
                              semi.at[r]).wait()

    NSUB = 4

    def start_gather(r, c, g):
        for s in range(NSUB):
            sub = pl.ds(s * (128 // NSUB), 128 // NSUB)
            pltpu.async_copy(cur_hbm.at[ebuf.at[r, c, 0, sub]],
                             rows.at[g, sub], semg.at[g])

    def wait_gather(g):
        for s in range(NSUB):
            sub = pl.ds(s * (128 // NSUB), 128 // NSUB)
            pltpu.make_async_copy(cur_hbm.at[ebuf.at[0, 0, 0, sub]],
                                  rows.at[g, sub], semg.at[g]).wait()

    def scale(r, c, g):
        def g_body(q, _):
            nv = nbuf[r, c, pl.ds(q * 16, 16)]
            for e in range(16):
                k = q * 16 + e
                s = _lane_bcast(nv, e)
                for f in range(8):
                    sl = pl.ds(f * 16, 16)
                    rows[g, k, sl] = rows[g, k, sl] * s
            return 0

        lax.fori_loop(0, 8, g_body, 0)

    pltpu.sync_copy(zeros_hbm, agg.at[pl.ds(sid * RPT, RPT)])
    refill(0, 0)
    wait_refill(0)
    start_gather(0, 0, 0)
    plsc.subcore_barrier()

    def iter_body(i, _):
        for jp in range(2):
            s, ns = jp, 1 - jp
            refill(i * 2 + jp + 1, ns)
            for c in range(SUP):
                p = (jp + c) % 2
                pn = 1 - p
                if c == SUP - 1:
                    wait_refill(ns)
                    start_gather(ns, 0, pn)
                else:
                    start_gather(s, c + 1, pn)
                wait_gather(p)
                scale(s, c, p)
                pltpu.sync_copy(rows.at[p], agg.at[ebuf.at[s, c, 1]],
                                add=True)
        return 0

    lax.fori_loop(0, NSUP // 2, iter_body, 0)

    wait_gather(0)
    plsc.subcore_barrier()

    sl = pl.ds(sid * RPT, RPT)
    pltpu.sync_copy(agg.at[sl], p_hbm.at[cid].at[sl])


_step = functools.partial(
    pl.kernel,
    out_type=jax.ShapeDtypeStruct((NC, N2, D), jnp.float32),
    mesh=plsc.VectorSubcoreMesh(core_axis_name="c", subcore_axis_name="s"),
    scratch_types=[
        pltpu.VMEM((2, SUP, 2, 128), jnp.int32),
        pltpu.VMEM((2, SUP, 128), jnp.float32),
        pltpu.VMEM((2, 128, D), jnp.float32),
        pltpu.VMEM_SHARED((N2, D), jnp.float32),
        pltpu.SemaphoreType.DMA((2,)),
        pltpu.SemaphoreType.DMA((2,)),
    ],
)(_step_body)


def _combine_body(p_ref, s_ref, cur_ref, so_ref):
    v = p_ref[0] + p_ref[1]
    cur_ref[...] = v
    so_ref[...] = s_ref[...] + v


def _combine(p, s):
    blk = 1024
    spec = pl.BlockSpec((blk, D), lambda i: (i, 0))
    return pl.pallas_call(
        _combine_body,
        grid=(N2 // blk,),
        in_specs=[pl.BlockSpec((NC, blk, D), lambda i: (0, i, 0)), spec],
        out_specs=(spec, spec),
        out_shape=(jax.ShapeDtypeStruct((N2, D), jnp.float32),
                   jax.ShapeDtypeStruct((N2, D), jnp.float32)),
    )(p, s)


def _final_body(x_ref, s_ref, wt_ref, b_ref, o_ref):
    h = ALPHA * x_ref[...] + COEF * s_ref[...]
    o_ref[...] = jnp.dot(h, wt_ref[...],
                         preferred_element_type=jnp.float32) + b_ref[...]


def _final(x, s, wt, b2):
    blk = 2000
    spec = pl.BlockSpec((blk, D), lambda i: (i, 0))
    return pl.pallas_call(
        _final_body,
        grid=(N // blk,),
        in_specs=[spec, spec,
                  pl.BlockSpec((D, D), lambda i: (0, 0)),
                  pl.BlockSpec((1, D), lambda i: (0, 0))],
        out_specs=spec,
        out_shape=jax.ShapeDtypeStruct((N, D), jnp.float32),
    )(x, s, wt, b2)


def kernel(x, edge_index, edge_weight, W, b):
    row, col = edge_index[0], edge_index[1]
    mask = row != col
    ew = jnp.where(mask, edge_weight, 0.0)
    loop_w = jnp.ones((N,), x.dtype).at[
        jnp.where(mask, N, row)].set(edge_weight, mode="drop")
    deg = jnp.zeros((N,), x.dtype).at[col].add(ew) + loop_w
    safe = deg > 0
    dinv = jnp.where(safe, lax.rsqrt(jnp.where(safe, deg, 1.0)), 0.0)
    nodes = jnp.arange(N, dtype=jnp.int32)
    pad = E2 - (E + N)
    npadc = NCPAD - NCHUNK
    zc_i = jnp.zeros((NW, npadc, 128), jnp.int32)
    zc_f = jnp.zeros((NW, npadc, 128), jnp.float32)
    rows_all = jnp.concatenate([
        jnp.concatenate([row, nodes, jnp.zeros((pad,), jnp.int32)]
                        ).reshape(NW, NCHUNK, 128), zc_i], axis=1)
    cols_all = jnp.concatenate([
        jnp.concatenate([col, nodes, jnp.zeros((pad,), jnp.int32)]
                        ).reshape(NW, NCHUNK, 128), zc_i], axis=1)
    ew_all = jnp.concatenate([
        jnp.concatenate([ew, loop_w, jnp.zeros((pad,), jnp.float32)]
                        ).reshape(NW, NCHUNK, 128), zc_f], axis=1)
    eidx = jnp.stack([rows_all, cols_all], axis=2).reshape(
        NW, NSPAD, SUP, 2, 128)
    dinv_pad = jnp.zeros((N2,), jnp.float32).at[:N].set(dinv)
    dinvrep = jnp.broadcast_to(dinv_pad[:, None], (N2, 16))

    nrm = _norm(dinvrep, eidx, ew_all).reshape(NW, NSPAD, SUP, 128)

    cur0 = jnp.zeros((N2, D), jnp.float32).at[:N].set(x)
    s0 = jnp.zeros((N2, D), jnp.float32)
    zeros = jnp.zeros((RPT, D), jnp.float32)

    def k_body(_, carry):
        cur, s = carry
        p = _step(cur, eidx, nrm, zeros)
        return _combine(p, s)

    _, s = lax.fori_loop(0, K, k_body, (cur0, s0))

    return _final(x, s[:N], W.T, b[None, :])


# speedup vs baseline: 2.1735x; 2.1735x over previous
import functools

import jax
import jax.numpy as jnp
from jax import lax
from jax.experimental import pallas as pl
from jax.experimental.pallas import tpu as pltpu
from jax.experimental.pallas import tpu_sc as plsc

N = 10000
E = 320000
D = 128
K = 16
ALPHA = 0.05
COEF = (1.0 - ALPHA) / K

NC = 2
NS = 16
NW = NC * NS

N2 = 10240
RPT = N2 // NS
NCHUNK = 81
EPT = NCHUNK * 128
E2 = NW * EPT

_GATHER_DNUMS = jax.lax.GatherDimensionNumbers(
    offset_dims=(), collapsed_slice_dims=(0,), start_index_map=(0,))


def _lane_bcast(v16, e):
    idx = jnp.full((16, 1), e, dtype=jnp.int32)
    return jax.lax.gather(v16, idx, _GATHER_DNUMS, (1,),
                          mode=jax.lax.GatherScatterMode.PROMISE_IN_BOUNDS)


def _onehot(e):
    lanes = lax.iota(jnp.int32, 16)
    return jnp.where(lanes == e, jnp.float32(1.0), jnp.float32(0.0))


def _norm_body(dinvrep_hbm, eidx_hbm, ew_hbm, nrm_hbm,
               ebuf, wbuf, dr, dc, nout, sem):
    cid = lax.axis_index("c")
    sid = lax.axis_index("s")
    wid = cid * NS + sid

    def chunk_body(j, _):
        pltpu.sync_copy(eidx_hbm.at[wid, j], ebuf)
        pltpu.sync_copy(ew_hbm.at[wid, j], wbuf)
        pltpu.async_copy(dinvrep_hbm.at[ebuf.at[0]], dr, sem).wait()
        pltpu.async_copy(dinvrep_hbm.at[ebuf.at[1]], dc, sem).wait()

        def g_body(q, _):
            sl = pl.ds(q * 16, 16)
            nv = jnp.zeros((16,), jnp.float32)
            for e in range(16):
                k = q * 16 + e
                nv = nv + dr[k, :] * dc[k, :] * _onehot(e)
            nout[sl] = nv * wbuf[sl]
            return 0

        lax.fori_loop(0, 8, g_body, 0)
        pltpu.sync_copy(nout, nrm_hbm.at[wid, j])
        return 0

    lax.fori_loop(0, NCHUNK, chunk_body, 0)


_norm = functools.partial(
    pl.kernel,
    out_type=jax.ShapeDtypeStruct((NW, NCHUNK, 128), jnp.float32),
    mesh=plsc.VectorSubcoreMesh(core_axis_name="c", subcore_axis_name="s"),
    compiler_params=pltpu.CompilerParams(use_tc_tiling_on_sc=False),
    scratch_types=[
        pltpu.VMEM((2, 128), jnp.int32),
        pltpu.VMEM((128,), jnp.float32),
        pltpu.VMEM((128, 16), jnp.float32),
        pltpu.VMEM((128, 16), jnp.float32),
        pltpu.VMEM((128,), jnp.float32),
        pltpu.SemaphoreType.DMA,
    ],
)(_norm_body)


def _step_body(cur_hbm, eidx_hbm, nrm_hbm, zeros_hbm,
               p0_hbm, p1_hbm,
               ebuf, nbuf, rowsv, agg, sem):
    cid = lax.axis_index("c")
    sid = lax.axis_index("s")
    wid = cid * NS + sid

    pltpu.sync_copy(zeros_hbm, agg.at[pl.ds(sid * RPT, RPT)])
    plsc.subcore_barrier()

    def chunk_body(j, _):
        pltpu.sync_copy(eidx_hbm.at[wid, j], ebuf)
        pltpu.sync_copy(nrm_hbm.at[wid, j], nbuf)
        pltpu.async_copy(cur_hbm.at[ebuf.at[0]], rowsv, sem).wait()

        def g_body(g, _):
            nv = nbuf[pl.ds(g * 16, 16)]
            for e in range(16):
                r = g * 16 + e
                s = _lane_bcast(nv, e)
                for f in range(8):
                    sl = pl.ds(f * 16, 16)
                    rowsv[r, sl] = rowsv[r, sl] * s
            return 0

        lax.fori_loop(0, 8, g_body, 0)

        pltpu.sync_copy(rowsv, agg.at[ebuf.at[1]], add=True)
        return 0

    lax.fori_loop(0, NCHUNK, chunk_body, 0)
    plsc.subcore_barrier()

    sl = pl.ds(sid * RPT, RPT)

    @pl.when(cid == 0)
    def _():
        pltpu.sync_copy(agg.at[sl], p0_hbm.at[sl])

    @pl.when(cid == 1)
    def _():
        pltpu.sync_copy(agg.at[sl], p1_hbm.at[sl])


_step = functools.partial(
    pl.kernel,
    out_type=(jax.ShapeDtypeStruct((N2, D), jnp.float32),
              jax.ShapeDtypeStruct((N2, D), jnp.float32)),
    mesh=plsc.VectorSubcoreMesh(core_axis_name="c", subcore_axis_name="s"),
    scratch_types=[
        pltpu.VMEM((2, 128), jnp.int32),
        pltpu.VMEM((128,), jnp.float32),
        pltpu.VMEM((128, D), jnp.float32),
        pltpu.VMEM_SHARED((N2, D), jnp.float32),
        pltpu.SemaphoreType.DMA,
    ],
)(_step_body)


def _combine_body(p0_ref, p1_ref, s_ref, cur_ref, so_ref):
    v = p0_ref[...] + p1_ref[...]
    cur_ref[...] = v
    so_ref[...] = s_ref[...] + v


def _combine(p0, p1, s):
    blk = 1024
    spec = pl.BlockSpec((blk, D), lambda i: (i, 0))
    return pl.pallas_call(
        _combine_body,
        grid=(N2 // blk,),
        in_specs=[spec, spec, spec],
        out_specs=(spec, spec),
        out_shape=(jax.ShapeDtypeStruct((N2, D), jnp.float32),
                   jax.ShapeDtypeStruct((N2, D), jnp.float32)),
    )(p0, p1, s)


def _final_body(x_ref, s_ref, wt_ref, b_ref, o_ref):
    h = ALPHA * x_ref[...] + COEF * s_ref[...]
    o_ref[...] = jnp.dot(h, wt_ref[...],
                         preferred_element_type=jnp.float32) + b_ref[...]


def _final(x, s, wt, b2):
    blk = 2000
    spec = pl.BlockSpec((blk, D), lambda i: (i, 0))
    return pl.pallas_call(
        _final_body,
        grid=(N // blk,),
        in_specs=[spec, spec,
                  pl.BlockSpec((D, D), lambda i: (0, 0)),
                  pl.BlockSpec((1, D), lambda i: (0, 0))],
        out_specs=spec,
        out_shape=jax.ShapeDtypeStruct((N, D), jnp.float32),
    )(x, s, wt, b2)


def kernel(x, edge_index, edge_weight, W, b):
    row, col = edge_index[0], edge_index[1]
    mask = row != col
    ew = jnp.where(mask, edge_weight, 0.0)
    loop_w = jnp.ones((N,), x.dtype).at[
        jnp.where(mask, N, row)].set(edge_weight, mode="drop")
    deg = jnp.zeros((N,), x.dtype).at[col].add(ew) + loop_w
    safe = deg > 0
    dinv = jnp.where(safe, lax.rsqrt(jnp.where(safe, deg, 1.0)), 0.0)

    nodes = jnp.arange(N, dtype=jnp.int32)
    pad = E2 - (E + N)
    rows_all = jnp.concatenate(
        [row, nodes, jnp.zeros((pad,), jnp.int32)]).reshape(NW, NCHUNK, 128)
    cols_all = jnp.concatenate(
        [col, nodes, jnp.zeros((pad,), jnp.int32)]).reshape(NW, NCHUNK, 128)
    ew_all = jnp.concatenate(
        [ew, loop_w, jnp.zeros((pad,), jnp.float32)]).reshape(NW, NCHUNK, 128)
    eidx = jnp.stack([rows_all, cols_all], axis=2)
    dinv_pad = jnp.zeros((N2,), jnp.float32).at[:N].set(dinv)
    dinvrep = jnp.broadcast_to(dinv_pad[:, None], (N2, 16))

    nrm = _norm(dinvrep, eidx, ew_all)

    cur0 = jnp.zeros((N2, D), jnp.float32).at[:N].set(x)
    s0 = jnp.zeros((N2, D), jnp.float32)
    zeros = jnp.zeros((RPT, D), jnp.float32)

    def k_body(_, carry):
        cur, s = carry
        p0, p1 = _step(cur, eidx, nrm, zeros)
        return _combine(p0, p1, s)

    _, s = lax.fori_loop(0, K, k_body, (cur0, s0))

    return _final(x, s[:N], W.T, b[None, :])
